# Initial kernel scaffold; baseline (speedup 1.0000x reference)
import jax
import jax.numpy as jnp
from jax.experimental import pallas as pl


def kernel(z_e, embedding):
    raise NotImplementedError("write your pallas kernel here")


# speedup vs baseline: 1.7205x; 1.7205x over previous
import functools

import jax
import jax.numpy as jnp
from jax.experimental import pallas as pl

NUM_CODES = 1024
DIM = 64
BETA = 0.25
ROWS = 2048


def _vq_block(z_ref, e_ref, zq_ref, idx_ref, loss_ref):
    i = pl.program_id(0)
    z = z_ref[...]
    e = e_ref[...]
    z_sq = jnp.sum(z * z, axis=1)
    e_sq = jnp.sum(e * e, axis=1)
    mm = jax.lax.dot_general(
        z, e, (((1,), (1,)), ((), ())),
        preferred_element_type=jnp.float32,
    )
    dist = (z_sq[:, None] + e_sq[None, :]) - 2.0 * mm
    minv = jnp.min(dist, axis=1)
    iota = jax.lax.broadcasted_iota(jnp.int32, dist.shape, 1)
    onehot_raw = dist == minv[:, None]
    cand = jnp.where(onehot_raw, iota, jnp.int32(NUM_CODES))
    idx = jnp.min(cand, axis=1)
    onehot = (iota == idx[:, None]).astype(jnp.float32)
    z_q = jax.lax.dot_general(
        onehot, e, (((1,), (0,)), ((), ())),
        preferred_element_type=jnp.float32,
    )
    diff = z_q - z
    zq_ref[...] = z + diff
    idx_ref[0, 0] = idx
    part = jnp.sum(diff * diff).reshape(1, 1)

    @pl.when(i == 0)
    def _init():
        loss_ref[...] = part

    @pl.when(i != 0)
    def _acc():
        loss_ref[...] += part


@functools.partial(jax.jit, static_argnames=("interpret",))
def kernel(z_e, embedding, interpret=False):
    B, D = z_e.shape[0], z_e.shape[1]
    spatial = z_e.shape[2:]
    ndim = z_e.ndim
    perm = (0,) + tuple(range(2, ndim)) + (1,)
    z_flat = jnp.transpose(z_e, perm).reshape(-1, D)
    n = z_flat.shape[0]
    nb = n // ROWS
    zq, idx, loss = pl.pallas_call(
        _vq_block,
        grid=(nb,),
        in_specs=[
            pl.BlockSpec((ROWS, D), lambda i: (i, 0)),
            pl.BlockSpec((NUM_CODES, D), lambda i: (0, 0)),
        ],
        out_specs=[
            pl.BlockSpec((ROWS, D), lambda i: (i, 0)),
            pl.BlockSpec((1, 1, ROWS), lambda i: (i, 0, 0)),
            pl.BlockSpec((1, 1), lambda i: (0, 0)),
        ],
        out_shape=[
            jax.ShapeDtypeStruct((n, D), jnp.float32),
            jax.ShapeDtypeStruct((nb, 1, ROWS), jnp.int32),
            jax.ShapeDtypeStruct((1, 1), jnp.float32),
        ],
        interpret=interpret,
    )(z_flat, embedding)
    inv_perm = (0, ndim - 1) + tuple(range(1, ndim - 1))
    z_q_st = jnp.transpose(zq.reshape((B,) + spatial + (D,)), inv_perm)
    indices_map = idx.reshape((B,) + spatial)
    codebook_loss = loss[0, 0] / (n * D)
    return (
        z_q_st,
        indices_map,
        (1.0 + BETA) * codebook_loss,
        codebook_loss,
        BETA * codebook_loss,
    )
